# Initial kernel scaffold; baseline (speedup 1.0000x reference)
import jax
import jax.numpy as jnp
from jax.experimental import pallas as pl


def kernel(id_, table):
    raise NotImplementedError("write your pallas kernel here")


# speedup vs baseline: 1.7623x; 1.7623x over previous
import jax
import jax.numpy as jnp
from jax import lax
from jax.experimental import pallas as pl
from jax.experimental.pallas import tpu as pltpu
from jax.experimental.pallas import tpu_sc as plsc

NUM_EMB = 1_000_000
D = 32
S = 8
L = 16
NC, NS = 2, 16
NW = NC * NS
CHUNK = 128
E_TOT = 4096 * 26
PER_W = E_TOT // NW
N_CHUNKS = PER_W // CHUNK


def _rsqrt_nr(ss):
    yi = jnp.int32(0x5F3759DF) - (plsc.bitcast(ss, jnp.int32) >> 1)
    y = plsc.bitcast(yi, jnp.float32)
    for _ in range(3):
        y = y * (1.5 - 0.5 * ss * y * y)
    return y


def _body(ids_hbm, table_hbm, out_hbm, ids_v, idx_v, rows_v, sums_v, sem):
    wid = lax.axis_index("s") * NC + lax.axis_index("c")
    base = wid * PER_W
    lane = lax.iota(jnp.int32, L)

    def do_chunk(k, carry):
        off = base + k * jnp.int32(CHUNK)
        pltpu.sync_copy(ids_hbm.at[pl.ds(off, CHUNK)], ids_v)
        for j in range(CHUNK // L):
            r = ids_v[pl.ds(j * L, L)] % NUM_EMB
            idx_v[jnp.int32(0), pl.ds(j * L, L)] = r
            for s in range(1, S):
                idx_v[jnp.int32(s), pl.ds(j * L, L)] = (r << s) % NUM_EMB
        copies = [
            pltpu.async_copy(table_hbm.at[idx_v.at[jnp.int32(s)]], rows_v.at[jnp.int32(s)], sem)
            for s in range(S)
        ]
        for c in copies:
            c.wait()

        def sum_one(e, c2):
            acc0 = rows_v[jnp.int32(0), e, pl.ds(0, L)]
            acc1 = rows_v[jnp.int32(0), e, pl.ds(L, L)]
            for s in range(1, S):
                acc0 = acc0 + rows_v[jnp.int32(s), e, pl.ds(0, L)]
                acc1 = acc1 + rows_v[jnp.int32(s), e, pl.ds(L, L)]
            sums_v[pl.ds(e * jnp.int32(D), L)] = acc0
            sums_v[pl.ds(e * jnp.int32(D) + jnp.int32(L), L)] = acc1
            return c2

        lax.fori_loop(jnp.int32(0), jnp.int32(CHUNK), sum_one, 0)

        for g in range(CHUNK // L):
            fbase = jnp.int32(g * L * D) + lane * jnp.int32(D)
            vals = []
            ss = jnp.zeros((L,), jnp.float32)
            for d in range(D):
                v = plsc.load_gather(sums_v, [fbase + jnp.int32(d)])
                vals.append(v)
                ss = ss + v * v
            inv = jnp.minimum(_rsqrt_nr(ss), 1e12)
            for d in range(D):
                plsc.store_scatter(
                    sums_v, [fbase + jnp.int32(d)], vals[d] * inv)
        pltpu.sync_copy(
            sums_v, out_hbm.at[pl.ds(off * jnp.int32(D), CHUNK * D)])
        return carry

    lax.fori_loop(jnp.int32(0), jnp.int32(N_CHUNKS), do_chunk, 0)


def kernel(id_, table):
    b, f = id_.shape
    ids32 = id_.reshape(-1).astype(jnp.int32)
    mesh = plsc.VectorSubcoreMesh(
        core_axis_name="c", subcore_axis_name="s",
        num_cores=NC, num_subcores=NS)
    out = pl.kernel(
        _body,
        out_type=jax.ShapeDtypeStruct((E_TOT * D,), jnp.float32),
        mesh=mesh,
        compiler_params=pltpu.CompilerParams(
            needs_layout_passes=False, use_tc_tiling_on_sc=False),
        scratch_types=[
            pltpu.VMEM((CHUNK,), jnp.int32),
            pltpu.VMEM((S, CHUNK), jnp.int32),
            pltpu.VMEM((S, CHUNK, D), jnp.float32),
            pltpu.VMEM((CHUNK * D,), jnp.float32),
            pltpu.SemaphoreType.DMA,
        ],
    )(ids32, table)
    return out.reshape(b, f, D)


# speedup vs baseline: 2.0815x; 1.1812x over previous
import jax
import jax.numpy as jnp
from jax import lax
from jax.experimental import pallas as pl
from jax.experimental.pallas import tpu as pltpu
from jax.experimental.pallas import tpu_sc as plsc

NUM_EMB = 1_000_000
D = 32
S = 8
L = 16
NC, NS = 2, 16
NW = NC * NS
CHUNK = 128
E_TOT = 4096 * 26
PER_W = E_TOT // NW
N_CHUNKS = PER_W // CHUNK


def _rsqrt_nr(ss):
    yi = jnp.int32(0x5F3759DF) - (plsc.bitcast(ss, jnp.int32) >> 1)
    y = plsc.bitcast(yi, jnp.float32)
    for _ in range(3):
        y = y * (1.5 - 0.5 * ss * y * y)
    return y


def _body(ids_hbm, table_hbm, out_hbm, ids_v, idx_v, rows_v, sem0, sem1):
    wid = lax.axis_index("s") * NC + lax.axis_index("c")
    base = wid * jnp.int32(PER_W)
    lane = lax.iota(jnp.int32, L)
    sems = (sem0, sem1)

    pltpu.sync_copy(ids_hbm.at[pl.ds(base, PER_W)], ids_v)

    def prep(k, p):
        pi = jnp.int32(p)

        def hash_one(j, c):
            jof = j * jnp.int32(L)
            r = ids_v[pl.ds(k * jnp.int32(CHUNK) + jof, L)]
            r = r % NUM_EMB

            def shift_one(s, rr):
                idx_v[pi, s, pl.ds(jof, L)] = rr
                return (rr << 1) % NUM_EMB

            lax.fori_loop(jnp.int32(0), jnp.int32(S), shift_one, r)
            return c

        lax.fori_loop(jnp.int32(0), jnp.int32(CHUNK // L), hash_one, 0)

        zero = jnp.zeros((L,), jnp.float32)

        def zero_one(e, c):
            rows_v[pi, e, pl.ds(0, L)] = zero
            rows_v[pi, e, pl.ds(L, L)] = zero
            return c

        lax.fori_loop(jnp.int32(0), jnp.int32(CHUNK), zero_one, 0)

        def fire_one(s, c):
            pltpu.async_copy(
                table_hbm.at[idx_v.at[pi, s]],
                rows_v.at[pi], sems[p], add=True)
            return c

        lax.fori_loop(jnp.int32(0), jnp.int32(S), fire_one, 0)

    def consume(k, p):
        pi = jnp.int32(p)

        def drain_one(s, c):
            pltpu.make_async_copy(
                table_hbm.at[idx_v.at[pi, s]],
                rows_v.at[pi], sems[p]).wait()
            return c

        lax.fori_loop(jnp.int32(0), jnp.int32(S), drain_one, 0)

        def norm_group(g, c):
            eidx = g * jnp.int32(L) + lane

            def acc_one(d, ss):
                dd = jnp.zeros((L,), jnp.int32) + d
                v = plsc.load_gather(rows_v.at[pi], [eidx, dd])
                return ss + v * v

            ss = lax.fori_loop(jnp.int32(0), jnp.int32(D), acc_one,
                               jnp.zeros((L,), jnp.float32))
            inv = jnp.minimum(_rsqrt_nr(ss), 1e12)

            def scale_one(d, c2):
                dd = jnp.zeros((L,), jnp.int32) + d
                v = plsc.load_gather(rows_v.at[pi], [eidx, dd])
                plsc.store_scatter(rows_v.at[pi], [eidx, dd], v * inv)
                return c2

            lax.fori_loop(jnp.int32(0), jnp.int32(D), scale_one, 0)
            return c

        lax.fori_loop(jnp.int32(0), jnp.int32(CHUNK // L), norm_group, 0)
        off = base + k * jnp.int32(CHUNK)
        pltpu.sync_copy(rows_v.at[pi], out_hbm.at[pl.ds(off, CHUNK)])

    prep(jnp.int32(0), 0)
    prep(jnp.int32(1), 1)

    def pair(i, c):
        k0 = i * jnp.int32(2)
        consume(k0, 0)

        @pl.when(k0 + 2 < N_CHUNKS)
        def _():
            prep(k0 + jnp.int32(2), 0)

        consume(k0 + jnp.int32(1), 1)

        @pl.when(k0 + 3 < N_CHUNKS)
        def _():
            prep(k0 + jnp.int32(3), 1)

        return c

    lax.fori_loop(jnp.int32(0), jnp.int32(N_CHUNKS // 2), pair, 0)


def kernel(id_, table):
    b, f = id_.shape
    ids32 = id_.reshape(-1).astype(jnp.int32)
    mesh = plsc.VectorSubcoreMesh(
        core_axis_name="c", subcore_axis_name="s",
        num_cores=NC, num_subcores=NS)
    out = pl.kernel(
        _body,
        out_type=jax.ShapeDtypeStruct((E_TOT, D), jnp.float32),
        mesh=mesh,
        compiler_params=pltpu.CompilerParams(
            needs_layout_passes=False, use_tc_tiling_on_sc=False),
        scratch_types=[
            pltpu.VMEM((PER_W,), jnp.int32),
            pltpu.VMEM((2, S, CHUNK), jnp.int32),
            pltpu.VMEM((2, CHUNK, D), jnp.float32),
            pltpu.SemaphoreType.DMA,
            pltpu.SemaphoreType.DMA,
        ],
    )(ids32, table)
    return out.reshape(b, f, D)
